# baseline (speedup 1.0000x reference)
import jax
import jax.numpy as jnp
from jax.experimental import pallas as pl


def _body(x_ref, adj_ref, w_ref, out_ref):
    b = x_ref.shape[0]
    n = x_ref.shape[1]
    w = w_ref[...]
    for i in range(b):
        a = adj_ref[i]
        xb = x_ref[i]
        t = jax.lax.dot_general(
            a, xb, (((0,), (0,)), ((), ())), preferred_element_type=jnp.float32
        )
        h = jnp.maximum(
            jax.lax.dot_general(
                t, w, (((1,), (0,)), ((), ())),
                preferred_element_type=jnp.float32,
            ),
            0.0,
        )
        out_ref[i, 0, :] = jnp.sum(h, axis=0) * (1.0 / n)


def kernel(x, adj, W):
    b, n, f_in = x.shape
    f_out = W.shape[1]
    return pl.pallas_call(
        _body,
        out_shape=jax.ShapeDtypeStruct((b, 1, f_out), jnp.float32),
    )(x, adj, W).reshape(b, f_out)
